# speedup vs baseline: 13.4467x; 13.4467x over previous
import functools

import jax
import jax.numpy as jnp
from jax import lax
from jax.experimental import pallas as pl
from jax.experimental.pallas import tpu as pltpu
from jax.experimental.pallas import tpu_sc as plsc

_LANES = 16


@functools.lru_cache(maxsize=None)
def _build(B, L, V, D, NC, NS):
    NW = NC * NS
    BW = B // NW
    IDXW = BW * L
    chunks = []
    off = 0
    while off < L:
        n = min(128, L - off)
        if L - off > 128:
            n = 104
        chunks.append((off, n))
        off += n
    ND = D // _LANES

    mesh = plsc.VectorSubcoreMesh(core_axis_name="c", subcore_axis_name="s")

    @functools.partial(
        pl.kernel,
        out_type=jax.ShapeDtypeStruct((B, D), jnp.float32),
        mesh=mesh,
        scratch_types=[
            pltpu.VMEM((IDXW,), jnp.int32),
            pltpu.VMEM((L, D), jnp.float32),
            pltpu.VMEM((L, D), jnp.float32),
            pltpu.VMEM((BW, D), jnp.float32),
            pltpu.SemaphoreType.DMA,
            pltpu.SemaphoreType.DMA,
        ],
    )
    def bow(x_hbm, w_hbm, out_hbm, idx_v, buf0, buf1, out_v, sem0, sem1):
        wid = lax.axis_index("s") * NC + lax.axis_index("c")
        pltpu.sync_copy(x_hbm.at[pl.ds(wid * IDXW, IDXW)], idx_v)

        def start(b, buf, sem):
            base = b * L
            for (o, n) in chunks:
                pltpu.async_copy(
                    w_hbm.at[idx_v.at[pl.ds(base + o, n)]],
                    buf.at[pl.ds(o, n)],
                    sem,
                )

        def drain(buf, sem):
            pltpu.make_async_copy(w_hbm.at[pl.ds(0, L)], buf, sem).wait()

        inv_l = jnp.float32(1.0 / L)

        def accumulate(b, buf):
            def rbody(r, accs):
                return tuple(
                    accs[j] + buf[r, pl.ds(j * _LANES, _LANES)]
                    for j in range(ND)
                )
            accs = lax.fori_loop(
                0, L, rbody,
                tuple(jnp.zeros((_LANES,), jnp.float32) for _ in range(ND)),
            )
            for j in range(ND):
                out_v[b, pl.ds(j * _LANES, _LANES)] = accs[j] * inv_l

        start(0, buf0, sem0)

        def gbody(g, carry):
            b0 = g * 2
            b1 = b0 + 1
            start(b1, buf1, sem1)
            drain(buf0, sem0)
            accumulate(b0, buf0)

            @pl.when(b1 + 1 < BW)
            def _():
                start(b1 + 1, buf0, sem0)

            drain(buf1, sem1)
            accumulate(b1, buf1)
            return carry

        lax.fori_loop(0, BW // 2, gbody, 0)
        pltpu.sync_copy(out_v, out_hbm.at[pl.ds(wid * BW, BW)])

    return bow


def kernel(x, weight):
    B, L = x.shape
    V, D = weight.shape
    info = plsc.get_sparse_core_info()
    bow = _build(B, L, V, D, info.num_cores, info.num_subcores)
    return bow(x.reshape(-1), weight)
